# baseline (speedup 1.0000x reference)
import functools

import numpy as np
import jax
import jax.numpy as jnp
from jax import lax
from jax.experimental import pallas as pl
from jax.experimental.pallas import tpu as pltpu

_MIN_GAP = 4
_MAX_GAP = 48
_REPEATED_SAMPLING = 4


def _temporal_indices(t: int):
    max_gap = min(_MAX_GAP, t - 1)
    gap = np.linspace(_MIN_GAP, max_gap, _REPEATED_SAMPLING).astype(np.int32)
    return [0] + [int(g) for g in gap]


def kernel(x):
    b, c, t, h, w = x.shape
    idx = _temporal_indices(t)
    k = len(idx)
    bc = b * c
    n_planes = bc * k
    nbuf = 8

    xv = x.reshape(bc * t, h, w)
    plan = [(p // k * t + idx[p % k], p) for p in range(n_planes)]

    def dma_body(x_hbm, o_hbm, *sc):
        bufs = sc[:nbuf]
        gsem = sc[nbuf:2 * nbuf]
        ssem = sc[2 * nbuf:3 * nbuf]
        gh = [None] * nbuf
        sh = [None] * nbuf
        dst_of = [None] * nbuf
        for i, (sp, dp) in enumerate(plan):
            bi = i % nbuf
            if i >= nbuf:
                sh[bi].wait()
            dst_of[bi] = dp
            gh[bi] = pltpu.make_async_copy(x_hbm.at[sp], bufs[bi], gsem[bi])
            gh[bi].start()
            pb = (i - 1) % nbuf
            if i >= 1:
                gh[pb].wait()
                sh[pb] = pltpu.make_async_copy(
                    bufs[pb], o_hbm.at[dst_of[pb]], ssem[pb])
                sh[pb].start()
        lb = (n_planes - 1) % nbuf
        gh[lb].wait()
        sh[lb] = pltpu.make_async_copy(bufs[lb], o_hbm.at[dst_of[lb]], ssem[lb])
        sh[lb].start()
        for bi in range(nbuf):
            sh[bi].wait()

    out = pl.pallas_call(
        dma_body,
        in_specs=[pl.BlockSpec(memory_space=pl.ANY)],
        out_specs=pl.BlockSpec(memory_space=pl.ANY),
        out_shape=jax.ShapeDtypeStruct((n_planes, h, w), x.dtype),
        scratch_shapes=(
            [pltpu.VMEM((h, w), x.dtype) for _ in range(nbuf)]
            + [pltpu.SemaphoreType.DMA for _ in range(2 * nbuf)]
        ),
    )(xv)
    return out.reshape(b, c, k, h, w)


# speedup vs baseline: 2.8330x; 2.8330x over previous
import functools

import numpy as np
import jax
import jax.numpy as jnp
from jax import lax
from jax.experimental import pallas as pl
from jax.experimental.pallas import tpu as pltpu

_MIN_GAP = 4
_MAX_GAP = 48
_REPEATED_SAMPLING = 4


def _temporal_indices(t: int):
    max_gap = min(_MAX_GAP, t - 1)
    gap = np.linspace(_MIN_GAP, max_gap, _REPEATED_SAMPLING).astype(np.int32)
    return [0] + [int(g) for g in gap]


def kernel(x):
    b, c, t, h, w = x.shape
    idx = _temporal_indices(t)
    k = len(idx)
    bc = b * c
    n_planes = bc * k
    nbuf = 16
    lag = 8

    xv = x.reshape(bc * t, h, w)
    plan = [(p // k * t + idx[p % k], p) for p in range(n_planes)]

    def dma_body(x_hbm, o_hbm, *sc):
        bufs = sc[:nbuf]
        gsem = sc[nbuf:2 * nbuf]
        ssem = sc[2 * nbuf:3 * nbuf]
        gh = {}
        sh = {}

        def start_gather(i):
            sp = plan[i][0]
            gh[i] = pltpu.make_async_copy(x_hbm.at[sp], bufs[i % nbuf], gsem[i % nbuf])
            gh[i].start()

        def start_scatter(i):
            dp = plan[i][1]
            sh[i] = pltpu.make_async_copy(bufs[i % nbuf], o_hbm.at[dp], ssem[i % nbuf])
            sh[i].start()

        for i in range(n_planes + lag):
            if i < n_planes:
                if i >= nbuf:
                    sh[i - nbuf].wait()
                start_gather(i)
            j = i - lag
            if 0 <= j < n_planes:
                gh[j].wait()
                start_scatter(j)
        for j in range(n_planes - nbuf, n_planes):
            sh[j].wait()

    out = pl.pallas_call(
        dma_body,
        in_specs=[pl.BlockSpec(memory_space=pl.ANY)],
        out_specs=pl.BlockSpec(memory_space=pl.ANY),
        out_shape=jax.ShapeDtypeStruct((n_planes, h, w), x.dtype),
        scratch_shapes=(
            [pltpu.VMEM((h, w), x.dtype) for _ in range(nbuf)]
            + [pltpu.SemaphoreType.DMA for _ in range(2 * nbuf)]
        ),
    )(xv)
    return out.reshape(b, c, k, h, w)


# speedup vs baseline: 3.0950x; 1.0925x over previous
import functools

import numpy as np
import jax
import jax.numpy as jnp
from jax import lax
from jax.experimental import pallas as pl
from jax.experimental.pallas import tpu as pltpu

_MIN_GAP = 4
_MAX_GAP = 48
_REPEATED_SAMPLING = 4


def _temporal_indices(t: int):
    max_gap = min(_MAX_GAP, t - 1)
    gap = np.linspace(_MIN_GAP, max_gap, _REPEATED_SAMPLING).astype(np.int32)
    return [0] + [int(g) for g in gap]


def kernel(x):
    b, c, t, h, w = x.shape
    idx = _temporal_indices(t)
    k = len(idx)
    bc = b * c
    n_planes = bc * k
    nbuf = 24
    lag = 12

    xv = x.reshape(bc * t, h, w)
    plan = [(p // k * t + idx[p % k], p) for p in range(n_planes)]

    def dma_body(x_hbm, o_hbm, *sc):
        bufs = sc[:nbuf]
        gsem = sc[nbuf:2 * nbuf]
        ssem = sc[2 * nbuf:3 * nbuf]
        gh = {}
        sh = {}

        def start_gather(i):
            sp = plan[i][0]
            gh[i] = pltpu.make_async_copy(x_hbm.at[sp], bufs[i % nbuf], gsem[i % nbuf])
            gh[i].start()

        def start_scatter(i):
            dp = plan[i][1]
            sh[i] = pltpu.make_async_copy(bufs[i % nbuf], o_hbm.at[dp], ssem[i % nbuf])
            sh[i].start()

        for i in range(n_planes + lag):
            if i < n_planes:
                if i >= nbuf:
                    sh[i - nbuf].wait()
                start_gather(i)
            j = i - lag
            if 0 <= j < n_planes:
                gh[j].wait()
                start_scatter(j)
        for j in range(n_planes - nbuf, n_planes):
            sh[j].wait()

    out = pl.pallas_call(
        dma_body,
        in_specs=[pl.BlockSpec(memory_space=pl.ANY)],
        out_specs=pl.BlockSpec(memory_space=pl.ANY),
        out_shape=jax.ShapeDtypeStruct((n_planes, h, w), x.dtype),
        scratch_shapes=(
            [pltpu.VMEM((h, w), x.dtype) for _ in range(nbuf)]
            + [pltpu.SemaphoreType.DMA for _ in range(2 * nbuf)]
        ),
    )(xv)
    return out.reshape(b, c, k, h, w)


# speedup vs baseline: 3.1911x; 1.0311x over previous
import functools

import numpy as np
import jax
import jax.numpy as jnp
from jax import lax
from jax.experimental import pallas as pl
from jax.experimental.pallas import tpu as pltpu

_MIN_GAP = 4
_MAX_GAP = 48
_REPEATED_SAMPLING = 4


def _temporal_indices(t: int):
    max_gap = min(_MAX_GAP, t - 1)
    gap = np.linspace(_MIN_GAP, max_gap, _REPEATED_SAMPLING).astype(np.int32)
    return [0] + [int(g) for g in gap]


def kernel(x):
    b, c, t, h, w = x.shape
    idx = _temporal_indices(t)
    k = len(idx)
    bc = b * c
    n_planes = bc * k
    nbuf = 32
    lag = 16

    xv = x.reshape(bc * t, h, w)
    plan = [(p // k * t + idx[p % k], p) for p in range(n_planes)]

    def dma_body(x_hbm, o_hbm, *sc):
        bufs = sc[:nbuf]
        gsem = sc[nbuf:2 * nbuf]
        ssem = sc[2 * nbuf:3 * nbuf]
        gh = {}
        sh = {}

        def start_gather(i):
            sp = plan[i][0]
            gh[i] = pltpu.make_async_copy(x_hbm.at[sp], bufs[i % nbuf], gsem[i % nbuf])
            gh[i].start()

        def start_scatter(i):
            dp = plan[i][1]
            sh[i] = pltpu.make_async_copy(bufs[i % nbuf], o_hbm.at[dp], ssem[i % nbuf])
            sh[i].start()

        for i in range(n_planes + lag):
            if i < n_planes:
                if i >= nbuf:
                    sh[i - nbuf].wait()
                start_gather(i)
            j = i - lag
            if 0 <= j < n_planes:
                gh[j].wait()
                start_scatter(j)
        for j in range(n_planes - nbuf, n_planes):
            sh[j].wait()

    out = pl.pallas_call(
        dma_body,
        in_specs=[pl.BlockSpec(memory_space=pl.ANY)],
        out_specs=pl.BlockSpec(memory_space=pl.ANY),
        out_shape=jax.ShapeDtypeStruct((n_planes, h, w), x.dtype),
        scratch_shapes=(
            [pltpu.VMEM((h, w), x.dtype) for _ in range(nbuf)]
            + [pltpu.SemaphoreType.DMA for _ in range(2 * nbuf)]
        ),
    )(xv)
    return out.reshape(b, c, k, h, w)
